# baseline (device time: 12530 ns/iter reference)
import jax
import jax.numpy as jnp
from jax import lax
from jax.experimental import pallas as pl
from jax.experimental.pallas import tpu as pltpu

N_DEV = 4
EPS = 1e-5


def kernel(x, t_emb, W_scale, W_shift):
    b, s, c = x.shape
    global_c = c * N_DEV

    def body(x_ref, t_ref, ws_ref, wsh_ref, out_ref, stats_ref, send_sems, recv_sems):
        my_pos = lax.axis_index("i")

        barrier_sem = pltpu.get_barrier_semaphore()
        for off in range(1, N_DEV):
            peer = (my_pos + off) % N_DEV
            pl.semaphore_signal(
                barrier_sem, inc=1,
                device_id=(peer,), device_id_type=pl.DeviceIdType.MESH,
            )
        pl.semaphore_wait(barrier_sem, N_DEV - 1)

        xv = x_ref[...]
        stats_ref[my_pos, 0] = jnp.sum(xv, axis=-1)
        stats_ref[my_pos, 1] = jnp.sum(xv * xv, axis=-1)

        sends = []
        for off in range(1, N_DEV):
            peer = (my_pos + off) % N_DEV
            rdma = pltpu.make_async_remote_copy(
                src_ref=stats_ref.at[my_pos],
                dst_ref=stats_ref.at[my_pos],
                send_sem=send_sems.at[off - 1],
                recv_sem=recv_sems.at[my_pos],
                device_id=(peer,),
                device_id_type=pl.DeviceIdType.MESH,
            )
            rdma.start()
            sends.append(rdma)

        scale = jnp.dot(t_ref[...], ws_ref[...], preferred_element_type=jnp.float32)
        shift = jnp.dot(t_ref[...], wsh_ref[...], preferred_element_type=jnp.float32)

        for off in range(1, N_DEV):
            src = (my_pos + off) % N_DEV
            recv = pltpu.make_async_remote_copy(
                src_ref=stats_ref.at[src],
                dst_ref=stats_ref.at[src],
                send_sem=send_sems.at[off - 1],
                recv_sem=recv_sems.at[src],
                device_id=(src,),
                device_id_type=pl.DeviceIdType.MESH,
            )
            recv.wait_recv()

        total = stats_ref[0] + stats_ref[1] + stats_ref[2] + stats_ref[3]
        mean = total[0] * (1.0 / global_c)
        var = total[1] * (1.0 / global_c) - mean * mean
        inv = lax.rsqrt(var + EPS)

        h = (xv - mean[:, :, None]) * inv[:, :, None]
        out_ref[...] = h * (1.0 + scale[:, None, :]) + shift[:, None, :]

        for rdma in sends:
            rdma.wait_send()

    return pl.pallas_call(
        body,
        out_shape=jax.ShapeDtypeStruct((b, s, c), jnp.float32),
        in_specs=[
            pl.BlockSpec(memory_space=pltpu.VMEM),
            pl.BlockSpec(memory_space=pltpu.VMEM),
            pl.BlockSpec(memory_space=pltpu.VMEM),
            pl.BlockSpec(memory_space=pltpu.VMEM),
        ],
        out_specs=pl.BlockSpec(memory_space=pltpu.VMEM),
        scratch_shapes=[
            pltpu.VMEM((N_DEV, 2, b, s), jnp.float32),
            pltpu.SemaphoreType.DMA((N_DEV - 1,)),
            pltpu.SemaphoreType.DMA((N_DEV,)),
        ],
        compiler_params=pltpu.CompilerParams(collective_id=0),
    )(x, t_emb, W_scale, W_shift)


# device time: 11833 ns/iter; 1.0589x vs baseline; 1.0589x over previous
import jax
import jax.numpy as jnp
from jax import lax
from jax.experimental import pallas as pl
from jax.experimental.pallas import tpu as pltpu

N_DEV = 4
EPS = 1e-5
K = 2


def kernel(x, t_emb, W_scale, W_shift):
    b, s, c = x.shape
    global_c = c * N_DEV
    sc = s // K

    def body(x_ref, t_ref, ws_ref, wsh_ref, out_ref, stats_ref, send_sems, recv_sems):
        my_pos = lax.axis_index("i")

        barrier_sem = pltpu.get_barrier_semaphore()
        for off in range(1, N_DEV):
            peer = (my_pos + off) % N_DEV
            pl.semaphore_signal(
                barrier_sem, inc=1,
                device_id=(peer,), device_id_type=pl.DeviceIdType.MESH,
            )

        def chunk_stats(k):
            xk = x_ref[:, pl.ds(k * sc, sc), :]
            stats_ref[k, my_pos, 0] = jnp.sum(xk, axis=-1)
            stats_ref[k, my_pos, 1] = jnp.sum(xk * xk, axis=-1)

        def start_sends(k):
            rdmas = []
            for off in range(1, N_DEV):
                peer = (my_pos + off) % N_DEV
                rdma = pltpu.make_async_remote_copy(
                    src_ref=stats_ref.at[k, my_pos],
                    dst_ref=stats_ref.at[k, my_pos],
                    send_sem=send_sems.at[k, off - 1],
                    recv_sem=recv_sems.at[k, my_pos],
                    device_id=(peer,),
                    device_id_type=pl.DeviceIdType.MESH,
                )
                rdma.start()
                rdmas.append(rdma)
            return rdmas

        chunk_stats(0)
        pl.semaphore_wait(barrier_sem, N_DEV - 1)
        sends = start_sends(0)
        for k in range(1, K):
            chunk_stats(k)
            sends += start_sends(k)

        scale = jnp.dot(t_ref[...], ws_ref[...], preferred_element_type=jnp.float32)
        shift = jnp.dot(t_ref[...], wsh_ref[...], preferred_element_type=jnp.float32)
        scale1 = 1.0 + scale[:, None, :]
        shift3 = shift[:, None, :]

        for k in range(K):
            for off in range(1, N_DEV):
                src = (my_pos + off) % N_DEV
                recv = pltpu.make_async_remote_copy(
                    src_ref=stats_ref.at[k, src],
                    dst_ref=stats_ref.at[k, src],
                    send_sem=send_sems.at[k, off - 1],
                    recv_sem=recv_sems.at[k, src],
                    device_id=(src,),
                    device_id_type=pl.DeviceIdType.MESH,
                )
                recv.wait_recv()

            total = (
                stats_ref[k, 0] + stats_ref[k, 1]
                + stats_ref[k, 2] + stats_ref[k, 3]
            )
            mean = total[0] * (1.0 / global_c)
            var = total[1] * (1.0 / global_c) - mean * mean
            inv = lax.rsqrt(var + EPS)

            xk = x_ref[:, pl.ds(k * sc, sc), :]
            h = (xk - mean[:, :, None]) * inv[:, :, None]
            out_ref[:, pl.ds(k * sc, sc), :] = h * scale1 + shift3

        for rdma in sends:
            rdma.wait_send()

    return pl.pallas_call(
        body,
        out_shape=jax.ShapeDtypeStruct((b, s, c), jnp.float32),
        in_specs=[
            pl.BlockSpec(memory_space=pltpu.VMEM),
            pl.BlockSpec(memory_space=pltpu.VMEM),
            pl.BlockSpec(memory_space=pltpu.VMEM),
            pl.BlockSpec(memory_space=pltpu.VMEM),
        ],
        out_specs=pl.BlockSpec(memory_space=pltpu.VMEM),
        scratch_shapes=[
            pltpu.VMEM((K, N_DEV, 2, b, sc), jnp.float32),
            pltpu.SemaphoreType.DMA((K, N_DEV - 1)),
            pltpu.SemaphoreType.DMA((K, N_DEV)),
        ],
        compiler_params=pltpu.CompilerParams(collective_id=0),
    )(x, t_emb, W_scale, W_shift)


# device time: 8524 ns/iter; 1.4700x vs baseline; 1.3882x over previous
import jax
import jax.numpy as jnp
from jax import lax
from jax.experimental import pallas as pl
from jax.experimental.pallas import tpu as pltpu

N_DEV = 4
EPS = 1e-5
PIECES = [
    (0, 4, 0, 128),
    (0, 4, 128, 256),
    (0, 4, 256, 384),
    (0, 4, 384, 512),
]
NP = len(PIECES)


def kernel(x, t_emb, W_scale, W_shift):
    x = pltpu.with_memory_space_constraint(x, pltpu.MemorySpace.HBM)
    t_emb = pltpu.with_memory_space_constraint(t_emb, pltpu.MemorySpace.HBM)
    W_scale = pltpu.with_memory_space_constraint(W_scale, pltpu.MemorySpace.HBM)
    W_shift = pltpu.with_memory_space_constraint(W_shift, pltpu.MemorySpace.HBM)
    b, s, c = x.shape
    d = t_emb.shape[1]
    global_c = c * N_DEV

    def body(*refs):
        (x_hbm, t_hbm, ws_hbm, wsh_hbm, out_ref,
         x_vmem, t_vmem, ws_vmem, wsh_vmem) = refs[:9]
        stats = refs[9:9 + NP]
        in_sems, w_sems, send_sems, recv_sems = refs[9 + NP:]
        my_pos = lax.axis_index("i")

        dma_x = []
        for p, (b0, b1, s0, s1) in enumerate(PIECES):
            cp = pltpu.make_async_copy(
                x_hbm.at[pl.ds(b0, b1 - b0), pl.ds(s0, s1 - s0), :],
                x_vmem.at[pl.ds(b0, b1 - b0), pl.ds(s0, s1 - s0), :],
                in_sems.at[p],
            )
            cp.start()
            dma_x.append(cp)
        dma_w = [
            pltpu.make_async_copy(t_hbm, t_vmem, w_sems.at[0]),
            pltpu.make_async_copy(ws_hbm, ws_vmem, w_sems.at[1]),
            pltpu.make_async_copy(wsh_hbm, wsh_vmem, w_sems.at[2]),
        ]
        for cp in dma_w:
            cp.start()

        barrier_sem = pltpu.get_barrier_semaphore()
        for off in range(1, N_DEV):
            peer = (my_pos + off) % N_DEV
            pl.semaphore_signal(
                barrier_sem, inc=1,
                device_id=(peer,), device_id_type=pl.DeviceIdType.MESH,
            )

        def start_sends(p):
            rdmas = []
            for off in (1, 2, 3):
                peer = (my_pos + off) % N_DEV
                rdma = pltpu.make_async_remote_copy(
                    src_ref=stats[p].at[my_pos],
                    dst_ref=stats[p].at[my_pos],
                    send_sem=send_sems.at[p, off - 1],
                    recv_sem=recv_sems.at[p, my_pos],
                    device_id=(peer,),
                    device_id_type=pl.DeviceIdType.MESH,
                )
                rdma.start()
                rdmas.append(rdma)
            return rdmas

        sends = []
        for p, (b0, b1, s0, s1) in enumerate(PIECES):
            dma_x[p].wait()
            xp = x_vmem[pl.ds(b0, b1 - b0), pl.ds(s0, s1 - s0), :]
            stats[p][my_pos, 0] = jnp.sum(xp, axis=-1)
            stats[p][my_pos, 1] = jnp.sum(xp * xp, axis=-1)
            if p == 0:
                pl.semaphore_wait(barrier_sem, N_DEV - 1)
            sends += start_sends(p)

        for cp in dma_w:
            cp.wait()
        scale = jnp.dot(t_vmem[...], ws_vmem[...], preferred_element_type=jnp.float32)
        shift = jnp.dot(t_vmem[...], wsh_vmem[...], preferred_element_type=jnp.float32)
        scale1 = 1.0 + scale[:, None, :]
        shift3 = shift[:, None, :]

        for p, (b0, b1, s0, s1) in enumerate(PIECES):
            for off in range(1, N_DEV):
                src = (my_pos + off) % N_DEV
                recv = pltpu.make_async_remote_copy(
                    src_ref=stats[p].at[src],
                    dst_ref=stats[p].at[src],
                    send_sem=send_sems.at[p, off - 1],
                    recv_sem=recv_sems.at[p, src],
                    device_id=(src,),
                    device_id_type=pl.DeviceIdType.MESH,
                )
                recv.wait_recv()

            total = stats[p][0] + stats[p][1] + stats[p][2] + stats[p][3]
            mean = total[0] * (1.0 / global_c)
            var = total[1] * (1.0 / global_c) - mean * mean
            inv = lax.rsqrt(var + EPS)

            xp = x_vmem[pl.ds(b0, b1 - b0), pl.ds(s0, s1 - s0), :]
            h = (xp - mean[:, :, None]) * inv[:, :, None]
            out_ref[pl.ds(b0, b1 - b0), pl.ds(s0, s1 - s0), :] = (
                h * scale1[b0:b1] + shift3[b0:b1]
            )

        for rdma in sends:
            rdma.wait_send()

    stats_shapes = [
        pltpu.VMEM((N_DEV, 2, b1 - b0, s1 - s0), jnp.float32)
        for (b0, b1, s0, s1) in PIECES
    ]
    return pl.pallas_call(
        body,
        out_shape=jax.ShapeDtypeStruct((b, s, c), jnp.float32),
        in_specs=[
            pl.BlockSpec(memory_space=pltpu.MemorySpace.HBM),
            pl.BlockSpec(memory_space=pltpu.MemorySpace.HBM),
            pl.BlockSpec(memory_space=pltpu.MemorySpace.HBM),
            pl.BlockSpec(memory_space=pltpu.MemorySpace.HBM),
        ],
        out_specs=pl.BlockSpec(memory_space=pltpu.VMEM),
        scratch_shapes=[
            pltpu.VMEM((b, s, c), jnp.float32),
            pltpu.VMEM((b, d), jnp.float32),
            pltpu.VMEM((d, c), jnp.float32),
            pltpu.VMEM((d, c), jnp.float32),
            *stats_shapes,
            pltpu.SemaphoreType.DMA((NP,)),
            pltpu.SemaphoreType.DMA((3,)),
            pltpu.SemaphoreType.DMA((NP, N_DEV - 1)),
            pltpu.SemaphoreType.DMA((NP, N_DEV)),
        ],
        compiler_params=pltpu.CompilerParams(collective_id=0),
    )(x, t_emb, W_scale, W_shift)
